# baseline (speedup 1.0000x reference)
import functools

import jax
import jax.numpy as jnp
from jax import lax
from jax.experimental import pallas as pl
from jax.experimental.pallas import tpu as pltpu
from jax.experimental.pallas import tpu_sc as plsc

N = 10000
E = 320000
D_IN = 128
D_HID = 128
D_OUT = 64

NUM_CORES = 2
NUM_SUBCORES = 16
NUM_TILES = NUM_CORES * NUM_SUBCORES
EPT = E // NUM_TILES
K = 80
NCH = EPT // K
RPT = N // NUM_SUBCORES
ZR = 125
WB = 624
WB_TAIL = N - NUM_SUBCORES * WB

_MESH = plsc.VectorSubcoreMesh(core_axis_name="c", subcore_axis_name="s")
_SC_PARAMS = pltpu.CompilerParams(use_tc_tiling_on_sc=False)


def _zero_acc(zbuf, acc, sid, d):
    def _zrow(i, carry):
        for j in range(d // 16):
            zbuf[i, pl.ds(j * 16, 16)] = jnp.zeros((16,), jnp.float32)
        return carry

    lax.fori_loop(0, ZR, _zrow, 0)
    for z in range(RPT // ZR):
        pltpu.sync_copy(zbuf, acc.at[pl.ds(sid * RPT + z * ZR, ZR)])


def _deg_body(col_hbm, ew_hbm, out_hbm, cidx, ewv, rows, zbuf, acc, sem):
    cid = lax.axis_index("c")
    sid = lax.axis_index("s")
    wid = cid * NUM_SUBCORES + sid

    _zero_acc(zbuf, acc, sid, 16)
    plsc.subcore_barrier()

    pltpu.sync_copy(col_hbm.at[wid], cidx)
    pltpu.sync_copy(ew_hbm.at[wid], ewv)

    def chunk(i, carry):
        def fill(g, c2):
            wv = ewv[i, pl.ds(g * 16, 16)]
            for j in range(16):
                rows[g * 16 + j, :] = jnp.full((16,), wv[j], jnp.float32)
            return c2

        lax.fori_loop(0, K // 16, fill, 0)
        pltpu.sync_copy(rows, acc.at[cidx.at[i]], add=True)
        return carry

    lax.fori_loop(0, NCH, chunk, 0)
    plsc.subcore_barrier()
    pltpu.sync_copy(acc.at[pl.ds(sid * WB, WB)],
                    out_hbm.at[cid, pl.ds(sid * WB, WB)])

    @pl.when(sid == 0)
    def _tail():
        pltpu.sync_copy(acc.at[pl.ds(NUM_SUBCORES * WB, WB_TAIL)],
                        out_hbm.at[cid, pl.ds(NUM_SUBCORES * WB, WB_TAIL)])


_deg_kernel = functools.partial(
    pl.kernel,
    out_type=jax.ShapeDtypeStruct((NUM_CORES, N, 16), jnp.float32),
    mesh=_MESH,
    scratch_types=[
        pltpu.VMEM((NCH, K), jnp.int32),
        pltpu.VMEM((NCH, K), jnp.float32),
        pltpu.VMEM((K, 16), jnp.float32),
        pltpu.VMEM((ZR, 16), jnp.float32),
        pltpu.VMEM_SHARED((N, 16), jnp.float32),
        pltpu.SemaphoreType.DMA,
    ],
    compiler_params=_SC_PARAMS,
)(_deg_body)


def _msg_body(d, y_hbm, row_hbm, col_hbm, ew_hbm, out_hbm,
              ridx, cidx, ewv, rows0, rows1, zbuf, acc,
              g0, g1, s0, s1):
    cid = lax.axis_index("c")
    sid = lax.axis_index("s")
    wid = cid * NUM_SUBCORES + sid

    _zero_acc(zbuf, acc, sid, d)
    plsc.subcore_barrier()

    pltpu.sync_copy(row_hbm.at[wid], ridx)
    pltpu.sync_copy(col_hbm.at[wid], cidx)
    pltpu.sync_copy(ew_hbm.at[wid], ewv)

    def gather(i, buf, sem):
        return pltpu.make_async_copy(y_hbm.at[ridx.at[i]], buf, sem)

    def scatter(i, buf, sem):
        return pltpu.make_async_copy(buf, acc.at[cidx.at[i]], sem)

    def scale(i, buf):
        def body(g, c2):
            wv = ewv[i, pl.ds(g * 16, 16)]
            for u in range(16):
                e = g * 16 + u
                w = wv[u]
                for j in range(d // 16):
                    buf[e, pl.ds(j * 16, 16)] = (
                        buf[e, pl.ds(j * 16, 16)] * w)
            return c2

        lax.fori_loop(0, K // 16, body, 0)

    gather(0, rows0, g0).start()
    gather(1, rows1, g1).start()

    def pair(t, carry):
        i0 = 2 * t
        gather(i0, rows0, g0).wait()
        scale(i0, rows0)
        scatter(i0, rows0, s0).start(add=True)
        gather(i0 + 1, rows1, g1).wait()
        scale(i0 + 1, rows1)
        scatter(i0 + 1, rows1, s1).start(add=True)
        scatter(i0, rows0, s0).wait()
        gather(i0 + 2, rows0, g0).start()
        scatter(i0 + 1, rows1, s1).wait()
        gather(i0 + 3, rows1, g1).start()
        return carry

    lax.fori_loop(0, (NCH - 3) // 2, pair, 0)
    i0 = NCH - 3
    gather(i0, rows0, g0).wait()
    scale(i0, rows0)
    scatter(i0, rows0, s0).start(add=True)
    gather(i0 + 1, rows1, g1).wait()
    scale(i0 + 1, rows1)
    scatter(i0 + 1, rows1, s1).start(add=True)
    scatter(i0, rows0, s0).wait()
    gather(i0 + 2, rows0, g0).start()
    gather(i0 + 2, rows0, g0).wait()
    scale(i0 + 2, rows0)
    scatter(i0 + 2, rows0, s0).start(add=True)
    scatter(i0 + 1, rows1, s1).wait()
    scatter(i0 + 2, rows0, s0).wait()
    plsc.subcore_barrier()
    pltpu.sync_copy(acc.at[pl.ds(sid * WB, WB)],
                    out_hbm.at[cid, pl.ds(sid * WB, WB)])

    @pl.when(sid == 0)
    def _tail():
        pltpu.sync_copy(acc.at[pl.ds(NUM_SUBCORES * WB, WB_TAIL)],
                        out_hbm.at[cid, pl.ds(NUM_SUBCORES * WB, WB_TAIL)])


def _make_msg_kernel(d):
    return functools.partial(
        pl.kernel,
        out_type=jax.ShapeDtypeStruct((NUM_CORES, N, d), jnp.float32),
        mesh=_MESH,
        scratch_types=[
            pltpu.VMEM((NCH, K), jnp.int32),
            pltpu.VMEM((NCH, K), jnp.int32),
            pltpu.VMEM((NCH, K), jnp.float32),
            pltpu.VMEM((K, d), jnp.float32),
            pltpu.VMEM((K, d), jnp.float32),
            pltpu.VMEM((ZR, d), jnp.float32),
            pltpu.VMEM_SHARED((N, d), jnp.float32),
            pltpu.SemaphoreType.DMA,
            pltpu.SemaphoreType.DMA,
            pltpu.SemaphoreType.DMA,
            pltpu.SemaphoreType.DMA,
        ],
        compiler_params=_SC_PARAMS,
    )(functools.partial(_msg_body, d))


_msg_kernel_64 = _make_msg_kernel(D_OUT)


_R = 1000
_G = N // _R


def _mm_body(x_ref, w_ref, o_ref):
    o_ref[...] = jnp.dot(x_ref[...], w_ref[...],
                         preferred_element_type=jnp.float32)


def _matmul(x, w):
    return pl.pallas_call(
        _mm_body,
        grid=(_G,),
        in_specs=[
            pl.BlockSpec((_R, x.shape[1]), lambda i: (i, 0)),
            pl.BlockSpec(w.shape, lambda i: (0, 0)),
        ],
        out_specs=pl.BlockSpec((_R, w.shape[1]), lambda i: (i, 0)),
        out_shape=jax.ShapeDtypeStruct((x.shape[0], w.shape[1]),
                                       jnp.float32),
    )(x, w)


def _dinv_of(degp_ref):
    deg = degp_ref[0, :, :1] + degp_ref[1, :, :1] + 1.0
    return lax.rsqrt(deg)


def _prescale_body(xw_ref, degp_ref, ya_ref, yb_ref):
    y = xw_ref[...] * _dinv_of(degp_ref)
    ya_ref[...] = y[:, :D_OUT]
    yb_ref[...] = y[:, D_OUT:]


def _prescale(xw, degp):
    d = xw.shape[1]
    return pl.pallas_call(
        _prescale_body,
        grid=(_G,),
        in_specs=[
            pl.BlockSpec((_R, d), lambda i: (i, 0)),
            pl.BlockSpec((NUM_CORES, _R, 16), lambda i: (0, i, 0)),
        ],
        out_specs=[
            pl.BlockSpec((_R, D_OUT), lambda i: (i, 0)),
            pl.BlockSpec((_R, D_OUT), lambda i: (i, 0)),
        ],
        out_shape=[
            jax.ShapeDtypeStruct((N, D_OUT), jnp.float32),
            jax.ShapeDtypeStruct((N, D_OUT), jnp.float32),
        ],
    )(xw, degp)


def _layer_body(aggpa_ref, aggpb_ref, xw_ref, degp_ref, b_ref, w_ref,
                xw2_ref, y2_ref):
    dinv = _dinv_of(degp_ref)
    agg = jnp.concatenate(
        [aggpa_ref[0] + aggpa_ref[1], aggpb_ref[0] + aggpb_ref[1]], axis=1)
    s = dinv * agg + (dinv * dinv) * xw_ref[...] + b_ref[...]
    h = jnp.maximum(s, 0.0)
    xw2 = jnp.dot(h, w_ref[...], preferred_element_type=jnp.float32)
    xw2_ref[...] = xw2
    y2_ref[...] = xw2 * dinv


def _layer(aggpa, aggpb, xw1, degp, b1, w2):
    d_in = xw1.shape[1]
    d_out = w2.shape[1]
    return pl.pallas_call(
        _layer_body,
        grid=(_G,),
        in_specs=[
            pl.BlockSpec((NUM_CORES, _R, D_OUT), lambda i: (0, i, 0)),
            pl.BlockSpec((NUM_CORES, _R, D_OUT), lambda i: (0, i, 0)),
            pl.BlockSpec((_R, d_in), lambda i: (i, 0)),
            pl.BlockSpec((NUM_CORES, _R, 16), lambda i: (0, i, 0)),
            pl.BlockSpec((1, d_in), lambda i: (0, 0)),
            pl.BlockSpec((d_in, d_out), lambda i: (0, 0)),
        ],
        out_specs=[
            pl.BlockSpec((_R, d_out), lambda i: (i, 0)),
            pl.BlockSpec((_R, d_out), lambda i: (i, 0)),
        ],
        out_shape=[
            jax.ShapeDtypeStruct((N, d_out), jnp.float32),
            jax.ShapeDtypeStruct((N, d_out), jnp.float32),
        ],
    )(aggpa, aggpb, xw1, degp, b1, w2)


def _final_body(aggp_ref, xw_ref, degp_ref, b_ref, o_ref):
    dinv = _dinv_of(degp_ref)
    o_ref[...] = (dinv * (aggp_ref[0] + aggp_ref[1])
                  + (dinv * dinv) * xw_ref[...] + b_ref[...])


def _final(aggp, xw2, degp, b2):
    d = xw2.shape[1]
    return pl.pallas_call(
        _final_body,
        grid=(_G,),
        in_specs=[
            pl.BlockSpec((NUM_CORES, _R, d), lambda i: (0, i, 0)),
            pl.BlockSpec((_R, d), lambda i: (i, 0)),
            pl.BlockSpec((NUM_CORES, _R, 16), lambda i: (0, i, 0)),
            pl.BlockSpec((1, d), lambda i: (0, 0)),
        ],
        out_specs=pl.BlockSpec((_R, d), lambda i: (i, 0)),
        out_shape=jax.ShapeDtypeStruct((N, d), jnp.float32),
    )(aggp, xw2, degp, b2)


def kernel(x, edge_index, edge_attr, W1, b1, W2, b2):
    row = edge_index[0].reshape(NUM_TILES, NCH, K)
    col = edge_index[1].reshape(NUM_TILES, NCH, K)
    ew = edge_attr.reshape(NUM_TILES, NCH, K)
    b1r = b1.reshape(1, D_HID)
    b2r = b2.reshape(1, D_OUT)

    degp = _deg_kernel(col, ew)
    xw1 = _matmul(x, W1)
    y1a, y1b = _prescale(xw1, degp)
    aggp1a = _msg_kernel_64(y1a, row, col, ew)
    aggp1b = _msg_kernel_64(y1b, row, col, ew)
    xw2, y2 = _layer(aggp1a, aggp1b, xw1, degp, b1r, W2)
    aggp2 = _msg_kernel_64(y2, row, col, ew)
    out = _final(aggp2, xw2, degp, b2r)
    return out


# speedup vs baseline: 1.2194x; 1.2194x over previous
import functools

import jax
import jax.numpy as jnp
from jax import lax
from jax.experimental import pallas as pl
from jax.experimental.pallas import tpu as pltpu
from jax.experimental.pallas import tpu_sc as plsc

N = 10000
E = 320000
D_IN = 128
D_HID = 128
D_OUT = 64

NUM_CORES = 2
NUM_SUBCORES = 16
NUM_TILES = NUM_CORES * NUM_SUBCORES
EPT = E // NUM_TILES
K = 80
NCH = EPT // K
RPT = N // NUM_SUBCORES
ZR = 125
WB = 624
WB_TAIL = N - NUM_SUBCORES * WB
NBUF = 4

_MESH = plsc.VectorSubcoreMesh(core_axis_name="c", subcore_axis_name="s")
_SC_PARAMS = pltpu.CompilerParams(use_tc_tiling_on_sc=False)


def _zero_acc(zbuf, acc, sid, d):
    def _zrow(i, carry):
        for j in range(d // 16):
            zbuf[i, pl.ds(j * 16, 16)] = jnp.zeros((16,), jnp.float32)
        return carry

    lax.fori_loop(0, ZR, _zrow, 0)
    for z in range(RPT // ZR):
        pltpu.sync_copy(zbuf, acc.at[pl.ds(sid * RPT + z * ZR, ZR)])


def _deg_body(col_hbm, ew_hbm, out_hbm, cidx, ewv, rows, zbuf, acc, sem):
    cid = lax.axis_index("c")
    sid = lax.axis_index("s")
    wid = cid * NUM_SUBCORES + sid

    _zero_acc(zbuf, acc, sid, 16)
    plsc.subcore_barrier()

    pltpu.sync_copy(col_hbm.at[wid], cidx)
    pltpu.sync_copy(ew_hbm.at[wid], ewv)

    def chunk(i, carry):
        def fill(g, c2):
            wv = ewv[i, pl.ds(g * 16, 16)]
            for j in range(16):
                rows[g * 16 + j, :] = jnp.full((16,), wv[j], jnp.float32)
            return c2

        lax.fori_loop(0, K // 16, fill, 0)
        pltpu.sync_copy(rows, acc.at[cidx.at[i]], add=True)
        return carry

    lax.fori_loop(0, NCH, chunk, 0)
    plsc.subcore_barrier()
    pltpu.sync_copy(acc.at[pl.ds(sid * WB, WB)],
                    out_hbm.at[cid, pl.ds(sid * WB, WB)])

    @pl.when(sid == 0)
    def _tail():
        pltpu.sync_copy(acc.at[pl.ds(NUM_SUBCORES * WB, WB_TAIL)],
                        out_hbm.at[cid, pl.ds(NUM_SUBCORES * WB, WB_TAIL)])


_deg_kernel = functools.partial(
    pl.kernel,
    out_type=jax.ShapeDtypeStruct((NUM_CORES, N, 16), jnp.float32),
    mesh=_MESH,
    scratch_types=[
        pltpu.VMEM((NCH, K), jnp.int32),
        pltpu.VMEM((NCH, K), jnp.float32),
        pltpu.VMEM((K, 16), jnp.float32),
        pltpu.VMEM((ZR, 16), jnp.float32),
        pltpu.VMEM_SHARED((N, 16), jnp.float32),
        pltpu.SemaphoreType.DMA,
    ],
    compiler_params=_SC_PARAMS,
)(_deg_body)


def _msg_body(d, y_hbm, row_hbm, col_hbm, ew_hbm, out_hbm,
              ridx, cidx, ewv, bufs, zbuf, acc, gsems, ssems):
    cid = lax.axis_index("c")
    sid = lax.axis_index("s")
    wid = cid * NUM_SUBCORES + sid

    _zero_acc(zbuf, acc, sid, d)
    plsc.subcore_barrier()

    pltpu.sync_copy(row_hbm.at[wid], ridx)
    pltpu.sync_copy(col_hbm.at[wid], cidx)
    pltpu.sync_copy(ew_hbm.at[wid], ewv)

    def gather(i, q):
        return pltpu.make_async_copy(y_hbm.at[ridx.at[i]], bufs[q], gsems[q])

    def scatter(i, q):
        return pltpu.make_async_copy(bufs[q], acc.at[cidx.at[i]], ssems[q])

    def scale(i, q):
        buf = bufs[q]

        def body(g, c2):
            wv = ewv[i, pl.ds(g * 16, 16)]
            for u in range(16):
                e = g * 16 + u
                w = wv[u]
                for j in range(d // 16):
                    buf[e, pl.ds(j * 16, 16)] = (
                        buf[e, pl.ds(j * 16, 16)] * w)
            return c2

        lax.fori_loop(0, K // 16, body, 0)

    pltpu.make_async_copy(zbuf.at[pl.ds(0, K)], acc.at[cidx.at[0]],
                          ssems[NBUF - 1]).start(add=True)
    for i in range(NBUF - 1):
        gather(i, i).start()

    def block(i, q, refill):
        gather(i, q).wait()
        scale(i, q)
        scatter(i, q).start(add=True)
        scatter(i - 1, (q - 1) % NBUF).wait()
        if refill:
            gather(i + NBUF - 1, (q - 1) % NBUF).start()

    def quad(t, carry):
        i0 = NBUF * t
        for q in range(NBUF):
            block(i0 + q, q, True)
        return carry

    nloop = (NCH - (NBUF + 1)) // NBUF
    lax.fori_loop(0, nloop, quad, 0)
    for i in range(NBUF * nloop, NCH):
        block(i, i % NBUF, i + NBUF - 1 < NCH)
    scatter(NCH - 1, (NCH - 1) % NBUF).wait()
    plsc.subcore_barrier()
    pltpu.sync_copy(acc.at[pl.ds(sid * WB, WB)],
                    out_hbm.at[cid, pl.ds(sid * WB, WB)])

    @pl.when(sid == 0)
    def _tail():
        pltpu.sync_copy(acc.at[pl.ds(NUM_SUBCORES * WB, WB_TAIL)],
                        out_hbm.at[cid, pl.ds(NUM_SUBCORES * WB, WB_TAIL)])


def _make_msg_kernel(d):
    return functools.partial(
        pl.kernel,
        out_type=jax.ShapeDtypeStruct((NUM_CORES, N, d), jnp.float32),
        mesh=_MESH,
        scratch_types=[
            pltpu.VMEM((NCH, K), jnp.int32),
            pltpu.VMEM((NCH, K), jnp.int32),
            pltpu.VMEM((NCH, K), jnp.float32),
            [pltpu.VMEM((K, d), jnp.float32) for _ in range(NBUF)],
            pltpu.VMEM((ZR, d), jnp.float32),
            pltpu.VMEM_SHARED((N, d), jnp.float32),
            [pltpu.SemaphoreType.DMA for _ in range(NBUF)],
            [pltpu.SemaphoreType.DMA for _ in range(NBUF)],
        ],
        compiler_params=_SC_PARAMS,
    )(functools.partial(_msg_body, d))


_msg_kernel_64 = _make_msg_kernel(D_OUT)


_R = 1000
_G = N // _R


def _mm_body(x_ref, w_ref, o_ref):
    o_ref[...] = jnp.dot(x_ref[...], w_ref[...],
                         preferred_element_type=jnp.float32)


def _matmul(x, w):
    return pl.pallas_call(
        _mm_body,
        grid=(_G,),
        in_specs=[
            pl.BlockSpec((_R, x.shape[1]), lambda i: (i, 0)),
            pl.BlockSpec(w.shape, lambda i: (0, 0)),
        ],
        out_specs=pl.BlockSpec((_R, w.shape[1]), lambda i: (i, 0)),
        out_shape=jax.ShapeDtypeStruct((x.shape[0], w.shape[1]),
                                       jnp.float32),
    )(x, w)


def _dinv_of(degp_ref):
    deg = degp_ref[0, :, :1] + degp_ref[1, :, :1] + 1.0
    return lax.rsqrt(deg)


def _prescale_body(xw_ref, degp_ref, ya_ref, yb_ref):
    y = xw_ref[...] * _dinv_of(degp_ref)
    ya_ref[...] = y[:, :D_OUT]
    yb_ref[...] = y[:, D_OUT:]


def _prescale(xw, degp):
    d = xw.shape[1]
    return pl.pallas_call(
        _prescale_body,
        grid=(_G,),
        in_specs=[
            pl.BlockSpec((_R, d), lambda i: (i, 0)),
            pl.BlockSpec((NUM_CORES, _R, 16), lambda i: (0, i, 0)),
        ],
        out_specs=[
            pl.BlockSpec((_R, D_OUT), lambda i: (i, 0)),
            pl.BlockSpec((_R, D_OUT), lambda i: (i, 0)),
        ],
        out_shape=[
            jax.ShapeDtypeStruct((N, D_OUT), jnp.float32),
            jax.ShapeDtypeStruct((N, D_OUT), jnp.float32),
        ],
    )(xw, degp)


def _layer_body(aggpa_ref, aggpb_ref, xw_ref, degp_ref, b_ref, w_ref,
                xw2_ref, y2_ref):
    dinv = _dinv_of(degp_ref)
    agg = jnp.concatenate(
        [aggpa_ref[0] + aggpa_ref[1], aggpb_ref[0] + aggpb_ref[1]], axis=1)
    s = dinv * agg + (dinv * dinv) * xw_ref[...] + b_ref[...]
    h = jnp.maximum(s, 0.0)
    xw2 = jnp.dot(h, w_ref[...], preferred_element_type=jnp.float32)
    xw2_ref[...] = xw2
    y2_ref[...] = xw2 * dinv


def _layer(aggpa, aggpb, xw1, degp, b1, w2):
    d_in = xw1.shape[1]
    d_out = w2.shape[1]
    return pl.pallas_call(
        _layer_body,
        grid=(_G,),
        in_specs=[
            pl.BlockSpec((NUM_CORES, _R, D_OUT), lambda i: (0, i, 0)),
            pl.BlockSpec((NUM_CORES, _R, D_OUT), lambda i: (0, i, 0)),
            pl.BlockSpec((_R, d_in), lambda i: (i, 0)),
            pl.BlockSpec((NUM_CORES, _R, 16), lambda i: (0, i, 0)),
            pl.BlockSpec((1, d_in), lambda i: (0, 0)),
            pl.BlockSpec((d_in, d_out), lambda i: (0, 0)),
        ],
        out_specs=[
            pl.BlockSpec((_R, d_out), lambda i: (i, 0)),
            pl.BlockSpec((_R, d_out), lambda i: (i, 0)),
        ],
        out_shape=[
            jax.ShapeDtypeStruct((N, d_out), jnp.float32),
            jax.ShapeDtypeStruct((N, d_out), jnp.float32),
        ],
    )(aggpa, aggpb, xw1, degp, b1, w2)


def _final_body(aggp_ref, xw_ref, degp_ref, b_ref, o_ref):
    dinv = _dinv_of(degp_ref)
    o_ref[...] = (dinv * (aggp_ref[0] + aggp_ref[1])
                  + (dinv * dinv) * xw_ref[...] + b_ref[...])


def _final(aggp, xw2, degp, b2):
    d = xw2.shape[1]
    return pl.pallas_call(
        _final_body,
        grid=(_G,),
        in_specs=[
            pl.BlockSpec((NUM_CORES, _R, d), lambda i: (0, i, 0)),
            pl.BlockSpec((_R, d), lambda i: (i, 0)),
            pl.BlockSpec((NUM_CORES, _R, 16), lambda i: (0, i, 0)),
            pl.BlockSpec((1, d), lambda i: (0, 0)),
        ],
        out_specs=pl.BlockSpec((_R, d), lambda i: (i, 0)),
        out_shape=jax.ShapeDtypeStruct((N, d), jnp.float32),
    )(aggp, xw2, degp, b2)


def kernel(x, edge_index, edge_attr, W1, b1, W2, b2):
    row = edge_index[0].reshape(NUM_TILES, NCH, K)
    col = edge_index[1].reshape(NUM_TILES, NCH, K)
    ew = edge_attr.reshape(NUM_TILES, NCH, K)
    b1r = b1.reshape(1, D_HID)
    b2r = b2.reshape(1, D_OUT)

    degp = _deg_kernel(col, ew)
    xw1 = _matmul(x, W1)
    y1a, y1b = _prescale(xw1, degp)
    aggp1a = _msg_kernel_64(y1a, row, col, ew)
    aggp1b = _msg_kernel_64(y1b, row, col, ew)
    xw2, y2 = _layer(aggp1a, aggp1b, xw1, degp, b1r, W2)
    aggp2 = _msg_kernel_64(y2, row, col, ew)
    out = _final(aggp2, xw2, degp, b2r)
    return out
